# Initial kernel scaffold; baseline (speedup 1.0000x reference)
import jax
import jax.numpy as jnp
from jax.experimental import pallas as pl


def kernel(x, router_logits, w1, w2):
    raise NotImplementedError("write your pallas kernel here")


# speedup vs baseline: 1.1685x; 1.1685x over previous
import functools

import jax
import jax.numpy as jnp
from jax.experimental import pallas as pl
from jax.experimental.pallas import tpu as pltpu

HIDDEN = 1024
N_EXPERTS = 8
INTER = 1408
TOP_K = 2
T = 4096

TILE_T = 512


def _routing_kernel(logits_ref, w_ref):
    logits = logits_ref[...]
    m = jnp.max(logits, axis=-1, keepdims=True)
    p = jnp.exp(logits - m)
    p = p / jnp.sum(p, axis=-1, keepdims=True)
    iota = jax.lax.broadcasted_iota(jnp.int32, p.shape, 1)
    p1 = jnp.max(p, axis=-1, keepdims=True)
    is1 = p == p1
    first1 = iota == jnp.min(jnp.where(is1, iota, N_EXPERTS), axis=-1, keepdims=True)
    p_wo = jnp.where(first1, -jnp.inf, p)
    p2 = jnp.max(p_wo, axis=-1, keepdims=True)
    is2 = p_wo == p2
    first2 = iota == jnp.min(jnp.where(is2, iota, N_EXPERTS), axis=-1, keepdims=True)
    denom = p1 + p2
    w_ref[...] = jnp.where(first1, p1 / denom, 0.0) + jnp.where(first2, p2 / denom, 0.0)


def _moe_dense_kernel(x_ref, wgt_ref, w1_ref, w2_ref, out_ref):
    e = pl.program_id(1)
    h = jnp.dot(x_ref[...], w1_ref[0].T, preferred_element_type=jnp.float32)
    h = h * jax.nn.sigmoid(h)
    y = jnp.dot(h, w2_ref[0].T, preferred_element_type=jnp.float32)
    wgt = wgt_ref[...]
    eiota = jax.lax.broadcasted_iota(jnp.int32, wgt.shape, 1)
    wcol = jnp.sum(jnp.where(eiota == e, wgt, 0.0), axis=1)
    y = y * wcol[:, None]

    @pl.when(e == 0)
    def _():
        out_ref[...] = y

    @pl.when(e > 0)
    def _():
        out_ref[...] += y


@jax.jit
def kernel(x, router_logits, w1, w2):
    wgt = pl.pallas_call(
        _routing_kernel,
        out_shape=jax.ShapeDtypeStruct((T, N_EXPERTS), jnp.float32),
    )(router_logits)

    out = pl.pallas_call(
        _moe_dense_kernel,
        grid=(T // TILE_T, N_EXPERTS),
        in_specs=[
            pl.BlockSpec((TILE_T, HIDDEN), lambda i, e: (i, 0)),
            pl.BlockSpec((TILE_T, N_EXPERTS), lambda i, e: (i, 0)),
            pl.BlockSpec((1, INTER, HIDDEN), lambda i, e: (e, 0, 0)),
            pl.BlockSpec((1, HIDDEN, INTER), lambda i, e: (e, 0, 0)),
        ],
        out_specs=pl.BlockSpec((TILE_T, HIDDEN), lambda i, e: (i, 0)),
        out_shape=jax.ShapeDtypeStruct((T, HIDDEN), jnp.float32),
    )(x, wgt, w1, w2)
    return out
